# baseline (speedup 1.0000x reference)
import jax
import jax.numpy as jnp
from jax.experimental import pallas as pl

C = 32
N = 32
L = 4096


def _fphi_kernel(x_ref, w_ref, b_ref, bias_ref, o_ref):
    xb = x_ref[...]
    wt = w_ref[...].T
    z = jnp.dot(xb, wt, preferred_element_type=jnp.float32)
    z = z + b_ref[...]
    z2 = (z * z).astype(jnp.bfloat16)
    rows = jax.lax.broadcasted_iota(jnp.int32, (N * C, N), 0)
    cols = jax.lax.broadcasted_iota(jnp.int32, (N * C, N), 1)
    s = (rows // C == cols).astype(jnp.bfloat16)
    gs = jnp.dot(z2, s, preferred_element_type=jnp.float32)
    r = jnp.sqrt(gs) + bias_ref[...]
    o_ref[...] = r.T


@jax.jit
def kernel(x, adj, W, b, bias):
    del adj
    x2 = x[0]
    b1 = b[None, :]
    bias1 = bias[None, :]
    oc = N * C

    out = pl.pallas_call(
        _fphi_kernel,
        grid=(1,),
        in_specs=[
            pl.BlockSpec((L, C), lambda i: (0, 0)),
            pl.BlockSpec((oc, C), lambda i: (0, 0)),
            pl.BlockSpec((1, oc), lambda i: (0, 0)),
            pl.BlockSpec((1, N), lambda i: (0, 0)),
        ],
        out_specs=pl.BlockSpec((N, L), lambda i: (0, 0)),
        out_shape=jax.ShapeDtypeStruct((N, L), jnp.float32),
    )(x2, W, b1, bias1)
    return out[None]
